# Initial kernel scaffold; baseline (speedup 1.0000x reference)
import jax
import jax.numpy as jnp
from jax.experimental import pallas as pl


def kernel(fc7, weight, nembedding, target):
    raise NotImplementedError("write your pallas kernel here")


# speedup vs baseline: 2.5927x; 2.5927x over previous
import math

import jax
import jax.numpy as jnp
from jax.experimental import pallas as pl
from jax.experimental.pallas import tpu as pltpu

B = 1024
V = 100000
CBLK = 2048
NBLK = (V + CBLK - 1) // CBLK
SCALE = 64.0
COS_M2 = math.cos(0.5)
SIN_M2 = math.sin(0.5)
NEG = -1e30


def _arc_kernel(tgt_ref, x_ref, out_ref, m_ref, s_ref, zy_ref):
    pid = pl.program_id(0)

    @pl.when(pid == 0)
    def _init():
        m_ref[:, :] = jnp.full((B, 1), NEG, jnp.float32)
        s_ref[:, :] = jnp.zeros((B, 1), jnp.float32)
        zy_ref[:, :] = jnp.zeros((B, 1), jnp.float32)

    x = x_ref[:, :]
    lanes = jax.lax.broadcasted_iota(jnp.int32, (B, CBLK), 1)
    limit = V - pid * CBLK
    xm = jnp.where(lanes < limit, x, NEG)

    m_old = m_ref[:, :]
    bm = jnp.max(xm, axis=1, keepdims=True)
    m_new = jnp.maximum(m_old, bm)
    e = jnp.exp(xm - m_new)
    s_ref[:, :] = s_ref[:, :] * jnp.exp(m_old - m_new) + jnp.sum(
        e, axis=1, keepdims=True
    )
    m_ref[:, :] = m_new

    rel = tgt_ref[:, :] - pid * CBLK
    zy_ref[:, :] = zy_ref[:, :] + jnp.sum(
        jnp.where(lanes == rel, x, 0.0), axis=1, keepdims=True
    )

    @pl.when(pid == NBLK - 1)
    def _fin():
        m = m_ref[:, :]
        s = s_ref[:, :]
        zy = zy_ref[:, :]
        c = zy * (1.0 / SCALE)
        new_zy = SCALE * (c * COS_M2 - jnp.sqrt(1.0 - c * c) * SIN_M2)
        m2 = jnp.maximum(m, new_zy)
        inner = s * jnp.exp(m - m2) - jnp.exp(zy - m2) + jnp.exp(new_zy - m2)
        lse = m2 + jnp.log(inner)
        out_ref[:, :] = jnp.sum(lse - new_zy, keepdims=True) * (1.0 / B)


def kernel(fc7, weight, nembedding, target):
    tgt2d = target.reshape(B, 1).astype(jnp.int32)
    out = pl.pallas_call(
        _arc_kernel,
        grid=(NBLK,),
        in_specs=[
            pl.BlockSpec((B, 1), lambda i: (0, 0)),
            pl.BlockSpec((B, CBLK), lambda i: (0, i)),
        ],
        out_specs=pl.BlockSpec((1, 1), lambda i: (0, 0)),
        out_shape=jax.ShapeDtypeStruct((1, 1), jnp.float32),
        scratch_shapes=[
            pltpu.VMEM((B, 1), jnp.float32),
            pltpu.VMEM((B, 1), jnp.float32),
            pltpu.VMEM((B, 1), jnp.float32),
        ],
    )(tgt2d, fc7)
    return out[0, 0]


# speedup vs baseline: 2.6109x; 1.0070x over previous
import math

import jax
import jax.numpy as jnp
from jax.experimental import pallas as pl
from jax.experimental.pallas import tpu as pltpu

B = 1024
V = 100000
CBLK = 4096
NBLK = (V + CBLK - 1) // CBLK
SCALE = 64.0
COS_M2 = math.cos(0.5)
SIN_M2 = math.sin(0.5)
NEG = -1e30


def _arc_kernel(tgt_ref, x_ref, out_ref, m_ref, s_ref, zy_ref):
    pid = pl.program_id(0)

    @pl.when(pid == 0)
    def _init():
        m_ref[:, :] = jnp.full((B, 1), NEG, jnp.float32)
        s_ref[:, :] = jnp.zeros((B, 1), jnp.float32)
        zy_ref[:, :] = jnp.zeros((B, 1), jnp.float32)

    x = x_ref[:, :]
    lanes = jax.lax.broadcasted_iota(jnp.int32, (B, CBLK), 1)
    limit = V - pid * CBLK
    xm = jnp.where(lanes < limit, x, NEG)

    m_old = m_ref[:, :]
    bm = jnp.max(xm, axis=1, keepdims=True)
    m_new = jnp.maximum(m_old, bm)
    e = jnp.exp(xm - m_new)
    s_ref[:, :] = s_ref[:, :] * jnp.exp(m_old - m_new) + jnp.sum(
        e, axis=1, keepdims=True
    )
    m_ref[:, :] = m_new

    rel = tgt_ref[:, :] - pid * CBLK
    zy_ref[:, :] = zy_ref[:, :] + jnp.sum(
        jnp.where(lanes == rel, x, 0.0), axis=1, keepdims=True
    )

    @pl.when(pid == NBLK - 1)
    def _fin():
        m = m_ref[:, :]
        s = s_ref[:, :]
        zy = zy_ref[:, :]
        c = zy * (1.0 / SCALE)
        new_zy = SCALE * (c * COS_M2 - jnp.sqrt(1.0 - c * c) * SIN_M2)
        m2 = jnp.maximum(m, new_zy)
        inner = s * jnp.exp(m - m2) - jnp.exp(zy - m2) + jnp.exp(new_zy - m2)
        lse = m2 + jnp.log(inner)
        out_ref[:, :] = jnp.sum(lse - new_zy, keepdims=True) * (1.0 / B)


def kernel(fc7, weight, nembedding, target):
    tgt2d = target.reshape(B, 1).astype(jnp.int32)
    out = pl.pallas_call(
        _arc_kernel,
        grid=(NBLK,),
        in_specs=[
            pl.BlockSpec((B, 1), lambda i: (0, 0)),
            pl.BlockSpec((B, CBLK), lambda i: (0, i)),
        ],
        out_specs=pl.BlockSpec((1, 1), lambda i: (0, 0)),
        out_shape=jax.ShapeDtypeStruct((1, 1), jnp.float32),
        scratch_shapes=[
            pltpu.VMEM((B, 1), jnp.float32),
            pltpu.VMEM((B, 1), jnp.float32),
            pltpu.VMEM((B, 1), jnp.float32),
        ],
    )(tgt2d, fc7)
    return out[0, 0]
